# Initial kernel scaffold; baseline (speedup 1.0000x reference)
import jax
import jax.numpy as jnp
from jax.experimental import pallas as pl


def kernel(x, theta, theta_energy):
    raise NotImplementedError("write your pallas kernel here")


# speedup vs baseline: 1.6599x; 1.6599x over previous
import functools

import jax
import jax.numpy as jnp
from jax.experimental import pallas as pl

_N_STEPS = 4
_ROWS_PER_BLOCK = 8


def _lsb_kernel(x_ref, ug_ref, ua_ref, theta_ref, te_ref, out_ref):
    R, D = x_ref.shape
    f32 = jnp.float32

    t = theta_ref[...]
    tmax = jnp.max(t, axis=-1, keepdims=True)
    et = jnp.exp(t - tmax)
    w = et / jnp.sum(et, axis=-1, keepdims=True)
    w0, w1, w2, w3 = (w[:, 0:1], w[:, 1:2], w[:, 2:3], w[:, 3:4])

    def balance(d):
        return (w0 * (d / (1.0 + d)) + w1 * jnp.sqrt(d)
                + w2 * jnp.minimum(1.0, d) + w3 * jnp.maximum(1.0, d))

    te = te_ref[...]
    d_plus = jnp.exp(te)
    d_minus = jnp.exp(-te)
    p_plus = balance(d_plus)
    p_minus = balance(d_minus)
    lp_plus = jnp.log(p_plus)
    lp_minus = jnp.log(p_minus)

    x = x_ref[...]
    xb_mask = x > 0.5
    S = jnp.sum(jnp.where(xb_mask, p_minus, p_plus), axis=-1, keepdims=True)

    iota = jax.lax.broadcasted_iota(jnp.int32, (R, D), 1)

    for i in range(_N_STEPS):
        u = ug_ref[i]
        g = -jnp.log(-jnp.log(u))
        lsel = jnp.where(xb_mask, lp_minus, lp_plus)
        z = lsel + g
        zmax = jnp.max(z, axis=-1, keepdims=True)
        idx = jnp.min(jnp.where(z >= zmax, iota, D), axis=-1, keepdims=True)
        m = (iota == idx).astype(f32)

        xb = jnp.sum(x * m, axis=-1, keepdims=True)
        te_i = jnp.sum(te * m, axis=-1, keepdims=True)
        s = 1.0 - 2.0 * xb
        m_term = s * te_i
        pf = balance(jnp.exp(m_term))
        pr = balance(jnp.exp(-m_term))
        S_r = S - pf + pr
        la = jnp.minimum(m_term + jnp.log(pr) - jnp.log(S_r)
                         - jnp.log(pf) + jnp.log(S), 0.0)
        acc = jnp.exp(la) > ua_ref[:, i:i + 1]

        flip = acc & (m > 0.5)
        x = jnp.where(flip, 1.0 - x, x)
        xb_mask = x > 0.5
        S = jnp.where(acc, S_r, S)

    out_ref[...] = x


def kernel(x, theta, theta_energy):
    B, D = x.shape
    key = jax.random.key(42)
    tiny = jnp.finfo(jnp.float32).tiny
    ugs, uas = [], []
    for i in range(_N_STEPS):
        kc, ka = jax.random.split(jax.random.fold_in(key, i))
        ugs.append(jax.random.uniform(kc, (B, D), jnp.float32,
                                      minval=tiny, maxval=1.0))
        uas.append(jax.random.uniform(ka, (B,), jnp.float32))
    ug = jnp.stack(ugs)
    ua = jnp.stack(uas, axis=1)

    R = _ROWS_PER_BLOCK
    grid = (B // R,)
    out = pl.pallas_call(
        _lsb_kernel,
        grid=grid,
        in_specs=[
            pl.BlockSpec((R, D), lambda i: (i, 0)),
            pl.BlockSpec((_N_STEPS, R, D), lambda i: (0, i, 0)),
            pl.BlockSpec((R, _N_STEPS), lambda i: (i, 0)),
            pl.BlockSpec((1, 4), lambda i: (0, 0)),
            pl.BlockSpec((1, D), lambda i: (0, 0)),
        ],
        out_specs=pl.BlockSpec((R, D), lambda i: (i, 0)),
        out_shape=jax.ShapeDtypeStruct((B, D), x.dtype),
    )(x, ug, ua, theta.reshape(1, 4), theta_energy.reshape(1, D))
    return out
